# baseline (speedup 1.0000x reference)
import functools

import jax
import jax.numpy as jnp
from jax import lax
from jax.experimental import pallas as pl
from jax.experimental.pallas import tpu as pltpu
from jax.experimental.pallas import tpu_sc as plsc

NC = 2
NS = 16
NW = NC * NS
CHUNK = 128


def _mm(a, w):
    return lax.dot_general(a, w, (((1,), (0,)), ((), ())),
                           preferred_element_type=jnp.float32)


def _make_deg(npad, chunks):
    mesh = plsc.VectorSubcoreMesh(core_axis_name="c", subcore_axis_name="s")
    rows_per_tile = npad // NS

    @functools.partial(
        pl.kernel,
        out_type=jax.ShapeDtypeStruct((NC, npad), jnp.float32),
        mesh=mesh,
        scratch_types=[
            pltpu.VMEM((chunks, CHUNK), jnp.int32),
            pltpu.VMEM((2, CHUNK), jnp.float32),
            pltpu.VMEM_SHARED((npad,), jnp.float32),
            pltpu.SemaphoreType.DMA,
        ],
    )
    def deg(dstw_hbm, out_hbm, dst_v, const_v, acc_sh, sem):
        c = lax.axis_index("c")
        s = lax.axis_index("s")
        w = s * NC + c
        pltpu.sync_copy(dstw_hbm.at[w], dst_v)
        for j in range(CHUNK // 16):
            const_v[0, pl.ds(j * 16, 16)] = jnp.zeros((16,), jnp.float32)
            const_v[1, pl.ds(j * 16, 16)] = jnp.ones((16,), jnp.float32)
        base = s * rows_per_tile
        full, rem = divmod(rows_per_tile, CHUNK)
        for k in range(full):
            pltpu.sync_copy(const_v.at[0], acc_sh.at[pl.ds(base + k * CHUNK, CHUNK)])
        if rem:
            pltpu.sync_copy(const_v.at[0, pl.ds(0, rem)],
                            acc_sh.at[pl.ds(base + full * CHUNK, rem)])
        plsc.subcore_barrier()

        def body(j, carry):
            pltpu.sync_copy(const_v.at[1], acc_sh.at[dst_v.at[j]], add=True)
            return carry

        lax.fori_loop(0, chunks, body, 0)
        plsc.subcore_barrier()
        pltpu.sync_copy(acc_sh.at[pl.ds(base, rows_per_tile)],
                        out_hbm.at[c, pl.ds(base, rows_per_tile)])

    return deg


def _make_prop(npad, chunks):
    mesh = plsc.VectorSubcoreMesh(core_axis_name="c", subcore_axis_name="s")
    rows_per_tile = npad // NS

    @functools.partial(
        pl.kernel,
        out_type=jax.ShapeDtypeStruct((NC, npad, 128), jnp.float32),
        mesh=mesh,
        scratch_types=[
            pltpu.VMEM((chunks // 2, CHUNK), jnp.int32),
            pltpu.VMEM((chunks // 2, CHUNK), jnp.int32),
            pltpu.VMEM((CHUNK, 128), jnp.float32),
            pltpu.VMEM((CHUNK, 128), jnp.float32),
            pltpu.VMEM_SHARED((npad, 128), jnp.float32),
            pltpu.SemaphoreType.DMA,
            pltpu.SemaphoreType.DMA,
            pltpu.SemaphoreType.DMA,
            pltpu.SemaphoreType.DMA,
            pltpu.SemaphoreType.DMA,
        ],
    )
    def prop(g_hbm, srcw_hbm, dstw_hbm, out_hbm, src_v, dst_v,
             rows0_v, rows1_v, acc_sh, gsem0, gsem1, ssem0, ssem1, isem):
        c = lax.axis_index("c")
        s = lax.axis_index("s")
        w = s * NC + c
        halfc = chunks // 2
        nhalf = halfc // 2

        pltpu.async_copy(srcw_hbm.at[w, pl.ds(0, halfc)], src_v, isem)
        pltpu.async_copy(dstw_hbm.at[w, pl.ds(0, halfc)], dst_v, isem)

        def zrow(i, carry):
            for j in range(128 // 16):
                rows0_v[i, pl.ds(j * 16, 16)] = jnp.zeros((16,), jnp.float32)
            return carry

        lax.fori_loop(0, CHUNK, zrow, 0)
        base = s * rows_per_tile
        for k in range(rows_per_tile // CHUNK):
            pltpu.sync_copy(rows0_v, acc_sh.at[pl.ds(base + k * CHUNK, CHUNK)])
        pltpu.make_async_copy(srcw_hbm.at[w, pl.ds(0, halfc)], src_v, isem).wait()
        pltpu.make_async_copy(dstw_hbm.at[w, pl.ds(0, halfc)], dst_v, isem).wait()
        pltpu.async_copy(g_hbm.at[src_v.at[0]], rows0_v, gsem0)
        pltpu.async_copy(g_hbm.at[src_v.at[1]], rows1_v, gsem1)
        plsc.subcore_barrier()

        for h in range(2):
            if h == 1:
                pltpu.sync_copy(srcw_hbm.at[w, pl.ds(halfc, halfc)], src_v)
                pltpu.sync_copy(dstw_hbm.at[w, pl.ds(halfc, halfc)], dst_v)
                pltpu.async_copy(g_hbm.at[src_v.at[0]], rows0_v, gsem0)
                pltpu.async_copy(g_hbm.at[src_v.at[1]], rows1_v, gsem1)

            def body(jj, carry):
                j0 = 2 * jj
                j1 = j0 + 1
                pltpu.make_async_copy(g_hbm.at[src_v.at[j0]], rows0_v, gsem0).wait()
                pltpu.async_copy(rows0_v, acc_sh.at[dst_v.at[j0]], ssem0, add=True)
                pltpu.make_async_copy(g_hbm.at[src_v.at[j1]], rows1_v, gsem1).wait()
                pltpu.async_copy(rows1_v, acc_sh.at[dst_v.at[j1]], ssem1, add=True)
                pltpu.make_async_copy(rows0_v, acc_sh.at[dst_v.at[j0]],
                                      ssem0).wait()

                @pl.when(jj < nhalf - 1)
                def _():
                    pltpu.async_copy(g_hbm.at[src_v.at[j0 + 2]], rows0_v, gsem0)

                pltpu.make_async_copy(rows1_v, acc_sh.at[dst_v.at[j1]],
                                      ssem1).wait()

                @pl.when(jj < nhalf - 1)
                def _():
                    pltpu.async_copy(g_hbm.at[src_v.at[j1 + 2]], rows1_v, gsem1)
                return carry

            lax.fori_loop(0, nhalf, body, 0)
        plsc.subcore_barrier()
        pltpu.sync_copy(acc_sh.at[pl.ds(base, rows_per_tile)],
                        out_hbm.at[c, pl.ds(base, rows_per_tile)])

    return prop


def _dinv(dref):
    return lax.rsqrt(dref[0] + dref[1] + 1.0)


def _f32(ref):
    return ref[...].astype(jnp.float32)


def _psum(pref, gref):
    return (pref[0].astype(jnp.float32) + pref[1].astype(jnp.float32)
            + _f32(gref))


def _stage_in(dref, fref, cref, wf_ref, wc_ref, out_ref):
    h = _mm(fref[...], wf_ref[...]) + _mm(cref[...], wc_ref[...])
    out_ref[...] = h * _dinv(dref)


def _stage_tanh_mm(dref, pref, gref, w_ref, b_ref, out_ref):
    dinv = _dinv(dref)
    t = jnp.tanh(_psum(pref, gref) * dinv + b_ref[...])
    out_ref[...] = _mm(t, w_ref[...]) * dinv


def _stage_latent(dref, pref, gref, nref, cref, wz_ref, wc_ref, bm_ref,
                  blv_ref, z_ref, mean_ref, logvar_ref, g5_ref):
    dinv = _dinv(dref)
    a = _psum(pref, gref) * dinv
    mean = a[:, :64] + bm_ref[...]
    logvar = a[:, 64:] + blv_ref[...]
    z = nref[...] * jnp.exp(0.5 * logvar) + mean
    z_ref[...] = z
    mean_ref[...] = mean
    logvar_ref[...] = logvar
    g5_ref[...] = (_mm(z, wz_ref[...]) + _mm(cref[...], wc_ref[...])) * dinv


def _stage_out(dref, pref, gref, b_ref, out_ref):
    out_ref[...] = _psum(pref, gref) * _dinv(dref) + b_ref[...]


def kernel(feature, condition, edge_index, noise, We1, be1, We2, be2, Wm, bm,
           Wlv, blv, Wd1, bd1, Wd2, bd2, Wd3, bd3):
    n, feat = feature.shape
    cond = condition.shape[1]
    lat = noise.shape[1]
    e = edge_index.shape[1]

    npad = -(-(n + 16) // 2048) * 2048
    chunks = -(-(-(-e // (NW * CHUNK))) // 4) * 4
    ep = NW * chunks * CHUNK
    R = npad // 16
    nb = npad // R

    src = edge_index[0].astype(jnp.int32)
    dst = edge_index[1].astype(jnp.int32)
    padi = jnp.arange(ep - e, dtype=jnp.int32)
    src_pad = (padi * 997) % n
    dst_pad = n + (padi % 16)
    srcw = jnp.concatenate([src, src_pad]).reshape(NW, chunks, CHUNK)
    dstw = jnp.concatenate([dst, dst_pad]).reshape(NW, chunks, CHUNK)

    fpad = jnp.pad(feature, ((0, npad - n), (0, 0)))
    cpad = jnp.pad(condition, ((0, npad - n), (0, 0)))
    npadded = jnp.pad(noise, ((0, npad - n), (0, 0)))

    We1f, We1c = We1[:feat], We1[feat:]
    Wd1z, Wd1c = Wd1[:lat], Wd1[lat:]
    Wml = jnp.concatenate([Wm, Wlv], axis=1)
    b2 = lambda b: b.reshape(1, -1)

    deg_fn = _make_deg(npad, chunks)
    prop_fn = _make_prop(npad, chunks)

    degp = deg_fn(dstw)
    deg3 = degp[:, :, None]

    r128 = pl.BlockSpec((R, 128), lambda i: (i, 0))
    r64 = pl.BlockSpec((R, 64), lambda i: (i, 0))
    rC = pl.BlockSpec((R, cond), lambda i: (i, 0))
    pspec = pl.BlockSpec((2, R, 128), lambda i: (0, i, 0))
    dspec = pl.BlockSpec((2, R, 1), lambda i: (0, i, 0))
    full = lambda *shape: pl.BlockSpec(shape, lambda i, _s=len(shape): (0,) * _s)
    f128 = jax.ShapeDtypeStruct((npad, 128), jnp.float32)
    f64 = jax.ShapeDtypeStruct((npad, 64), jnp.float32)

    g1 = pl.pallas_call(
        _stage_in, grid=(nb,),
        in_specs=[dspec, r128, rC, full(feat, 128), full(cond, 128)],
        out_specs=r128, out_shape=f128,
    )(deg3, fpad, cpad, We1f, We1c)

    p = prop_fn(g1, srcw, dstw)

    def tanh_mm(p, g, W, b, wdim):
        return pl.pallas_call(
            _stage_tanh_mm, grid=(nb,),
            in_specs=[dspec, pspec, r128, full(128, wdim), full(1, 128)],
            out_specs=pl.BlockSpec((R, wdim), lambda i: (i, 0)),
            out_shape=jax.ShapeDtypeStruct((npad, wdim), jnp.float32),
        )(deg3, p, g, W, b)

    g2 = tanh_mm(p, g1, We2, b2(be1), 128)
    p = prop_fn(g2, srcw, dstw)
    g3 = tanh_mm(p, g2, Wml, b2(be2), 128)
    p = prop_fn(g3, srcw, dstw)

    z, mean, logvar, g5 = pl.pallas_call(
        _stage_latent, grid=(nb,),
        in_specs=[dspec, pspec, r128, r64, rC, full(lat, 128), full(cond, 128),
                  full(1, 64), full(1, 64)],
        out_specs=(r64, r64, r64, r128),
        out_shape=(f64, f64, f64, f128),
    )(deg3, p, g3, npadded, cpad, Wd1z, Wd1c, b2(bm), b2(blv))

    p = prop_fn(g5, srcw, dstw)
    g6 = tanh_mm(p, g5, Wd2, b2(bd1), 128)
    p = prop_fn(g6, srcw, dstw)
    g7 = tanh_mm(p, g6, Wd3, b2(bd2), 128)
    p = prop_fn(g7, srcw, dstw)

    out = pl.pallas_call(
        _stage_out, grid=(nb,),
        in_specs=[dspec, pspec, r128, full(1, 128)],
        out_specs=r128,
        out_shape=jax.ShapeDtypeStruct((npad, 128), jnp.float32),
    )(deg3, p, g7, b2(bd3))

    return z[:n], mean[:n], logvar[:n], out[:n]


# speedup vs baseline: 1.2650x; 1.2650x over previous
import functools

import jax
import jax.numpy as jnp
from jax import lax
from jax.experimental import pallas as pl
from jax.experimental.pallas import tpu as pltpu
from jax.experimental.pallas import tpu_sc as plsc

NC = 2
NS = 16
NW = NC * NS
CHUNK = 128


def _mm(a, w):
    return lax.dot_general(a, w, (((1,), (0,)), ((), ())),
                           preferred_element_type=jnp.float32)


def _make_deg(npad, chunks):
    mesh = plsc.VectorSubcoreMesh(core_axis_name="c", subcore_axis_name="s")
    rows_per_tile = npad // NS

    @functools.partial(
        pl.kernel,
        out_type=jax.ShapeDtypeStruct((NC, npad), jnp.float32),
        mesh=mesh,
        scratch_types=[
            pltpu.VMEM((chunks, CHUNK), jnp.int32),
            pltpu.VMEM((2, CHUNK), jnp.float32),
            pltpu.VMEM_SHARED((npad,), jnp.float32),
            pltpu.SemaphoreType.DMA,
        ],
    )
    def deg(dstw_hbm, out_hbm, dst_v, const_v, acc_sh, sem):
        c = lax.axis_index("c")
        s = lax.axis_index("s")
        w = s * NC + c
        pltpu.sync_copy(dstw_hbm.at[w], dst_v)
        for j in range(CHUNK // 16):
            const_v[0, pl.ds(j * 16, 16)] = jnp.zeros((16,), jnp.float32)
            const_v[1, pl.ds(j * 16, 16)] = jnp.ones((16,), jnp.float32)
        base = s * rows_per_tile
        full, rem = divmod(rows_per_tile, CHUNK)
        for k in range(full):
            pltpu.sync_copy(const_v.at[0], acc_sh.at[pl.ds(base + k * CHUNK, CHUNK)])
        if rem:
            pltpu.sync_copy(const_v.at[0, pl.ds(0, rem)],
                            acc_sh.at[pl.ds(base + full * CHUNK, rem)])
        plsc.subcore_barrier()

        def body(j, carry):
            pltpu.sync_copy(const_v.at[1], acc_sh.at[dst_v.at[j]], add=True)
            return carry

        lax.fori_loop(0, chunks, body, 0)
        plsc.subcore_barrier()
        pltpu.sync_copy(acc_sh.at[pl.ds(base, rows_per_tile)],
                        out_hbm.at[c, pl.ds(base, rows_per_tile)])

    return deg


def _make_prop(npad, chunks):
    mesh = plsc.VectorSubcoreMesh(core_axis_name="c", subcore_axis_name="s")
    rows_per_tile = npad // NS

    @functools.partial(
        pl.kernel,
        out_type=jax.ShapeDtypeStruct((NC, npad, 128), jnp.float32),
        mesh=mesh,
        scratch_types=[
            pltpu.VMEM((chunks // 2, CHUNK), jnp.int32),
            pltpu.VMEM((chunks // 2, CHUNK), jnp.int32),
            pltpu.VMEM((CHUNK, 128), jnp.float32),
            pltpu.VMEM((CHUNK, 128), jnp.float32),
            pltpu.VMEM_SHARED((npad, 128), jnp.float32),
            pltpu.SemaphoreType.DMA,
            pltpu.SemaphoreType.DMA,
            pltpu.SemaphoreType.DMA,
            pltpu.SemaphoreType.DMA,
            pltpu.SemaphoreType.DMA,
        ],
    )
    def prop(g_hbm, srcw_hbm, dstw_hbm, out_hbm, src_v, dst_v,
             rows0_v, rows1_v, acc_sh, gsem0, gsem1, ssem0, ssem1, isem):
        c = lax.axis_index("c")
        s = lax.axis_index("s")
        w = s * NC + c
        halfc = chunks // 2
        nhalf = halfc // 2

        pltpu.async_copy(srcw_hbm.at[w, pl.ds(0, halfc)], src_v, isem)
        pltpu.async_copy(dstw_hbm.at[w, pl.ds(0, halfc)], dst_v, isem)

        def zrow(i, carry):
            for j in range(128 // 16):
                rows0_v[i, pl.ds(j * 16, 16)] = jnp.zeros((16,), jnp.float32)
            return carry

        lax.fori_loop(0, CHUNK, zrow, 0)
        base = s * rows_per_tile
        for k in range(rows_per_tile // CHUNK):
            pltpu.sync_copy(rows0_v, acc_sh.at[pl.ds(base + k * CHUNK, CHUNK)])
        pltpu.make_async_copy(srcw_hbm.at[w, pl.ds(0, halfc)], src_v, isem).wait()
        pltpu.make_async_copy(dstw_hbm.at[w, pl.ds(0, halfc)], dst_v, isem).wait()
        pltpu.async_copy(g_hbm.at[src_v.at[0]], rows0_v, gsem0)
        pltpu.async_copy(g_hbm.at[src_v.at[1]], rows1_v, gsem1)
        plsc.subcore_barrier()

        for h in range(2):
            if h == 1:
                pltpu.sync_copy(srcw_hbm.at[w, pl.ds(halfc, halfc)], src_v)
                pltpu.sync_copy(dstw_hbm.at[w, pl.ds(halfc, halfc)], dst_v)
                pltpu.async_copy(g_hbm.at[src_v.at[0]], rows0_v, gsem0)
                pltpu.async_copy(g_hbm.at[src_v.at[1]], rows1_v, gsem1)

            def body(jj, carry):
                j0 = 2 * jj
                for j, buf, sem in ((j0, rows0_v, gsem0), (j0 + 1, rows1_v, gsem1)):
                    pltpu.make_async_copy(g_hbm.at[src_v.at[j]], buf, sem).wait()
                    pltpu.sync_copy(buf, acc_sh.at[dst_v.at[j]], add=True)

                    @pl.when(jj < nhalf - 1)
                    def _():
                        pltpu.async_copy(g_hbm.at[src_v.at[j + 2]], buf, sem)
                return carry

            lax.fori_loop(0, nhalf, body, 0)
        plsc.subcore_barrier()
        pltpu.sync_copy(acc_sh.at[pl.ds(base, rows_per_tile)],
                        out_hbm.at[c, pl.ds(base, rows_per_tile)])

    return prop


def _dinv(dref):
    return lax.rsqrt(dref[0] + dref[1] + 1.0)


def _f32(ref):
    return ref[...].astype(jnp.float32)


def _psum(pref, gref):
    return (pref[0].astype(jnp.float32) + pref[1].astype(jnp.float32)
            + _f32(gref))


def _stage_in(dref, fref, cref, wf_ref, wc_ref, out_ref):
    h = _mm(fref[...], wf_ref[...]) + _mm(cref[...], wc_ref[...])
    out_ref[...] = h * _dinv(dref)


def _stage_tanh_mm(dref, pref, gref, w_ref, b_ref, out_ref):
    dinv = _dinv(dref)
    t = jnp.tanh(_psum(pref, gref) * dinv + b_ref[...])
    out_ref[...] = _mm(t, w_ref[...]) * dinv


def _stage_latent(dref, pref, gref, nref, cref, wz_ref, wc_ref, bm_ref,
                  blv_ref, z_ref, mean_ref, logvar_ref, g5_ref):
    dinv = _dinv(dref)
    a = _psum(pref, gref) * dinv
    mean = a[:, :64] + bm_ref[...]
    logvar = a[:, 64:] + blv_ref[...]
    z = nref[...] * jnp.exp(0.5 * logvar) + mean
    z_ref[...] = z
    mean_ref[...] = mean
    logvar_ref[...] = logvar
    g5_ref[...] = (_mm(z, wz_ref[...]) + _mm(cref[...], wc_ref[...])) * dinv


def _stage_out(dref, pref, gref, b_ref, out_ref):
    out_ref[...] = _psum(pref, gref) * _dinv(dref) + b_ref[...]


def kernel(feature, condition, edge_index, noise, We1, be1, We2, be2, Wm, bm,
           Wlv, blv, Wd1, bd1, Wd2, bd2, Wd3, bd3):
    n, feat = feature.shape
    cond = condition.shape[1]
    lat = noise.shape[1]
    e = edge_index.shape[1]

    npad = -(-(n + 16) // 2048) * 2048
    chunks = -(-(-(-e // (NW * CHUNK))) // 4) * 4
    ep = NW * chunks * CHUNK
    R = npad // 16
    nb = npad // R

    src = edge_index[0].astype(jnp.int32)
    dst = edge_index[1].astype(jnp.int32)
    padi = jnp.arange(ep - e, dtype=jnp.int32)
    src_pad = (padi * 997) % n
    dst_pad = n + (padi % 16)
    srcw = jnp.concatenate([src, src_pad]).reshape(NW, chunks, CHUNK)
    dstw = jnp.concatenate([dst, dst_pad]).reshape(NW, chunks, CHUNK)

    fpad = jnp.pad(feature, ((0, npad - n), (0, 0)))
    cpad = jnp.pad(condition, ((0, npad - n), (0, 0)))
    npadded = jnp.pad(noise, ((0, npad - n), (0, 0)))

    We1f, We1c = We1[:feat], We1[feat:]
    Wd1z, Wd1c = Wd1[:lat], Wd1[lat:]
    Wml = jnp.concatenate([Wm, Wlv], axis=1)
    b2 = lambda b: b.reshape(1, -1)

    deg_fn = _make_deg(npad, chunks)
    prop_fn = _make_prop(npad, chunks)

    degp = deg_fn(dstw)
    deg3 = degp[:, :, None]

    r128 = pl.BlockSpec((R, 128), lambda i: (i, 0))
    r64 = pl.BlockSpec((R, 64), lambda i: (i, 0))
    rC = pl.BlockSpec((R, cond), lambda i: (i, 0))
    pspec = pl.BlockSpec((2, R, 128), lambda i: (0, i, 0))
    dspec = pl.BlockSpec((2, R, 1), lambda i: (0, i, 0))
    full = lambda *shape: pl.BlockSpec(shape, lambda i, _s=len(shape): (0,) * _s)
    f128 = jax.ShapeDtypeStruct((npad, 128), jnp.float32)
    f64 = jax.ShapeDtypeStruct((npad, 64), jnp.float32)

    g1 = pl.pallas_call(
        _stage_in, grid=(nb,),
        in_specs=[dspec, r128, rC, full(feat, 128), full(cond, 128)],
        out_specs=r128, out_shape=f128,
    )(deg3, fpad, cpad, We1f, We1c)

    p = prop_fn(g1, srcw, dstw)

    def tanh_mm(p, g, W, b, wdim):
        return pl.pallas_call(
            _stage_tanh_mm, grid=(nb,),
            in_specs=[dspec, pspec, r128, full(128, wdim), full(1, 128)],
            out_specs=pl.BlockSpec((R, wdim), lambda i: (i, 0)),
            out_shape=jax.ShapeDtypeStruct((npad, wdim), jnp.float32),
        )(deg3, p, g, W, b)

    g2 = tanh_mm(p, g1, We2, b2(be1), 128)
    p = prop_fn(g2, srcw, dstw)
    g3 = tanh_mm(p, g2, Wml, b2(be2), 128)
    p = prop_fn(g3, srcw, dstw)

    z, mean, logvar, g5 = pl.pallas_call(
        _stage_latent, grid=(nb,),
        in_specs=[dspec, pspec, r128, r64, rC, full(lat, 128), full(cond, 128),
                  full(1, 64), full(1, 64)],
        out_specs=(r64, r64, r64, r128),
        out_shape=(f64, f64, f64, f128),
    )(deg3, p, g3, npadded, cpad, Wd1z, Wd1c, b2(bm), b2(blv))

    p = prop_fn(g5, srcw, dstw)
    g6 = tanh_mm(p, g5, Wd2, b2(bd1), 128)
    p = prop_fn(g6, srcw, dstw)
    g7 = tanh_mm(p, g6, Wd3, b2(bd2), 128)
    p = prop_fn(g7, srcw, dstw)

    out = pl.pallas_call(
        _stage_out, grid=(nb,),
        in_specs=[dspec, pspec, r128, full(1, 128)],
        out_specs=r128,
        out_shape=jax.ShapeDtypeStruct((npad, 128), jnp.float32),
    )(deg3, p, g7, b2(bd3))

    return z[:n], mean[:n], logvar[:n], out[:n]
